# baseline (speedup 1.0000x reference)
import functools

import jax
import jax.numpy as jnp
from jax import lax
from jax.experimental import pallas as pl
from jax.experimental.pallas import tpu as pltpu
from jax.experimental.pallas import tpu_sc as plsc

N_NODES = 10000
N_EDGES = 320000
CH = 128

NC = 2
NS = 16
NW = NC * NS
ECH = 128
NCHUNK = 79
EPW = NCHUNK * ECH
E_PAD = NW * EPW
N_TRASH = 64
N_ACC = N_NODES + N_TRASH
RPT = 624
RPT_LAST = N_NODES - RPT * (NS - 1)

_MESH = plsc.VectorSubcoreMesh(
    core_axis_name="c", subcore_axis_name="s", num_cores=NC, num_subcores=NS)


def _deg_body(dst_hbm, ones_hbm, zeros_hbm, deg_out,
              dst_v, ones_v, deg_sh):
  c = lax.axis_index("c")
  s = lax.axis_index("s")
  w = s * NC + c

  @pl.when(s == 0)
  def _():
    pltpu.sync_copy(zeros_hbm, deg_sh)

  pltpu.sync_copy(dst_hbm.at[w], dst_v)
  pltpu.sync_copy(ones_hbm, ones_v)
  plsc.subcore_barrier()

  @pl.loop(0, NCHUNK)
  def _(j):
    pltpu.sync_copy(ones_v, deg_sh.at[dst_v.at[j]], add=True)

  plsc.subcore_barrier()

  @pl.when(s == 0)
  def _():
    pltpu.sync_copy(deg_sh, deg_out.at[c])


_deg_kernel = functools.partial(
    pl.kernel,
    out_type=jax.ShapeDtypeStruct((NC, N_ACC), jnp.float32),
    mesh=_MESH,
    scratch_types=[
        pltpu.VMEM((NCHUNK, ECH), jnp.int32),
        pltpu.VMEM((ECH,), jnp.float32),
        pltpu.VMEM_SHARED((N_ACC,), jnp.float32),
    ],
)(_deg_body)


def _edge_body(g_hbm, src_hbm, dst_hbm, zrow_hbm, out_hbm,
               idx_v, b0, b1, b2, acc, is0, is1, is2, g0, g1, g2):
  c = lax.axis_index("c")
  s = lax.axis_index("s")
  w = s * NC + c
  base = w * EPW
  isems = (is0, is1, is2)
  bufs = (b0, b1, b2)
  gsems = (g0, g1, g2)

  def fetch_idx(j, k):
    pltpu.async_copy(src_hbm.at[pl.ds(base + j * ECH, ECH)], idx_v.at[k],
                     isems[k])
    pltpu.async_copy(dst_hbm.at[pl.ds(base + j * ECH, ECH)], idx_v.at[4 + k],
                     isems[k])

  def wait_idx(j, k):
    pltpu.make_async_copy(
        src_hbm.at[pl.ds(base + j * ECH, ECH)], idx_v.at[k], isems[k]).wait()
    pltpu.make_async_copy(
        dst_hbm.at[pl.ds(base + j * ECH, ECH)], idx_v.at[4 + k],
        isems[k]).wait()

  def start_gather(m):
    pltpu.async_copy(g_hbm.at[idx_v.at[m]], bufs[m], gsems[m])

  def wait_gather(m):
    pltpu.make_async_copy(g_hbm.at[idx_v.at[m]], bufs[m], gsems[m]).wait()

  def scatter(m):
    pltpu.sync_copy(bufs[m], acc.at[idx_v.at[4 + m]], add=True)

  for k in range(3):
    fetch_idx(k, k)

  @pl.when(s < NS - 1)
  def _():
    pltpu.sync_copy(zrow_hbm.at[pl.ds(s * RPT, RPT)],
                    acc.at[pl.ds(s * RPT, RPT)])

  @pl.when(s == NS - 1)
  def _():
    pltpu.sync_copy(zrow_hbm.at[pl.ds((NS - 1) * RPT, RPT_LAST)],
                    acc.at[pl.ds((NS - 1) * RPT, RPT_LAST)])

  plsc.subcore_barrier()
  wait_idx(0, 0)
  start_gather(0)
  wait_idx(1, 1)
  start_gather(1)

  def step(c, m):
    wait_gather(m)
    m2 = (m + 2) % 3
    wait_idx(c + 2, m2)
    start_gather(m2)
    scatter(m)
    fetch_idx(c + 3, m)

  @pl.loop(0, NCHUNK - 4, step=3)
  def _(j):
    for m in range(3):
      step(j + m, m)

  wait_gather(0)
  wait_idx(NCHUNK - 2, 2)
  start_gather(2)
  scatter(0)
  fetch_idx(NCHUNK - 1, 0)
  wait_gather(1)
  wait_idx(NCHUNK - 1, 0)
  start_gather(0)
  scatter(1)
  wait_gather(2)
  scatter(2)
  wait_gather(0)
  scatter(0)

  plsc.subcore_barrier()

  @pl.when(s < NS - 1)
  def _():
    pltpu.sync_copy(acc.at[pl.ds(s * RPT, RPT)],
                    out_hbm.at[c, pl.ds(s * RPT, RPT)])

  @pl.when(s == NS - 1)
  def _():
    pltpu.sync_copy(acc.at[pl.ds((NS - 1) * RPT, RPT_LAST)],
                    out_hbm.at[c, pl.ds((NS - 1) * RPT, RPT_LAST)])


_edge_kernel = functools.partial(
    pl.kernel,
    out_type=jax.ShapeDtypeStruct((NC, N_NODES, CH), jnp.float32),
    mesh=_MESH,
    scratch_types=[
        pltpu.VMEM((8, ECH), jnp.int32),
        pltpu.VMEM((ECH, CH), jnp.float32),
        pltpu.VMEM((ECH, CH), jnp.float32),
        pltpu.VMEM((ECH, CH), jnp.float32),
        pltpu.VMEM_SHARED((N_ACC, CH), jnp.float32),
        pltpu.SemaphoreType.DMA,
        pltpu.SemaphoreType.DMA,
        pltpu.SemaphoreType.DMA,
        pltpu.SemaphoreType.DMA,
        pltpu.SemaphoreType.DMA,
        pltpu.SemaphoreType.DMA,
    ],
)(_edge_body)


BM = 1000

def _dinv_block(deg_ref):
  dl = deg_ref[0]
  return lax.rsqrt(dl[0] + dl[1] + 1.0)


def _lin_body(deg_ref, x_ref, w_ref, g_ref):
  dinv = _dinv_block(deg_ref)
  h = jnp.dot(x_ref[...], w_ref[...], preferred_element_type=jnp.float32)
  g_ref[...] = h * dinv[:, None]


def _lin(deg2, x, W):
  return pl.pallas_call(
      _lin_body,
      grid=(N_NODES // BM,),
      in_specs=[
          pl.BlockSpec((1, NC, BM), lambda i: (i, 0, 0)),
          pl.BlockSpec((BM, CH), lambda i: (i, 0)),
          pl.BlockSpec((CH, CH), lambda i: (0, 0)),
      ],
      out_specs=pl.BlockSpec((BM, CH), lambda i: (i, 0)),
      out_shape=jax.ShapeDtypeStruct((N_NODES, CH), jnp.float32),
  )(deg2, x, W)


def _fin_body(deg_ref, p_ref, g_ref, b_ref, o_ref):
  dinv = _dinv_block(deg_ref)
  t = (p_ref[0] + p_ref[1] + g_ref[...]) * dinv[:, None] + b_ref[...]
  o_ref[...] = jnp.maximum(t, 0.0)


def _fin(deg2, P, g, b2):
  return pl.pallas_call(
      _fin_body,
      grid=(N_NODES // BM,),
      in_specs=[
          pl.BlockSpec((1, NC, BM), lambda i: (i, 0, 0)),
          pl.BlockSpec((NC, BM, CH), lambda i: (0, i, 0)),
          pl.BlockSpec((BM, CH), lambda i: (i, 0)),
          pl.BlockSpec((1, CH), lambda i: (0, 0)),
      ],
      out_specs=pl.BlockSpec((BM, CH), lambda i: (i, 0)),
      out_shape=jax.ShapeDtypeStruct((N_NODES, CH), jnp.float32),
  )(deg2, P, g, b2)


@jax.jit
def kernel(x, edge_index, W, b):
  npad = E_PAD - N_EDGES
  src = jnp.concatenate(
      [edge_index[0].astype(jnp.int32),
       jnp.arange(npad, dtype=jnp.int32) % N_NODES])
  dst = jnp.concatenate(
      [edge_index[1].astype(jnp.int32),
       N_NODES + (jnp.arange(npad, dtype=jnp.int32) % N_TRASH)])
  ones_c = jnp.ones((ECH,), jnp.float32)
  zeros_n = jnp.zeros((N_ACC,), jnp.float32)
  zrow = jnp.zeros((N_NODES, CH), jnp.float32)

  deg2 = _deg_kernel(dst.reshape(NW, NCHUNK, ECH), ones_c, zeros_n)[:, :N_NODES]
  deg2 = deg2.reshape(NC, N_NODES // BM, BM).transpose(1, 0, 2)
  g = _lin(deg2, x, W)
  P = _edge_kernel(g, src, dst, zrow)
  return _fin(deg2, P, g, b.reshape(1, CH))


# speedup vs baseline: 1.1140x; 1.1140x over previous
import functools

import jax
import jax.numpy as jnp
from jax import lax
from jax.experimental import pallas as pl
from jax.experimental.pallas import tpu as pltpu
from jax.experimental.pallas import tpu_sc as plsc

N_NODES = 10000
N_EDGES = 320000
CH = 128

NC = 2
NS = 16
NW = NC * NS
ECH = 128
NCHUNK = 79
EPW = NCHUNK * ECH
E_PAD = NW * EPW
N_TRASH = 64
N_ACC = N_NODES + N_TRASH
RPT = 624
RPT_LAST = N_NODES - RPT * (NS - 1)

_MESH = plsc.VectorSubcoreMesh(
    core_axis_name="c", subcore_axis_name="s", num_cores=NC, num_subcores=NS)


def _deg_body(dst_hbm, ones_hbm, zeros_hbm, deg_out,
              dst_v, ones_v, deg_sh, dsem):
  c = lax.axis_index("c")
  s = lax.axis_index("s")
  w = s * NC + c

  @pl.when(s == 0)
  def _():
    pltpu.sync_copy(zeros_hbm, deg_sh)

  pltpu.sync_copy(dst_hbm.at[w], dst_v)
  pltpu.sync_copy(ones_hbm, ones_v)
  plsc.subcore_barrier()

  @pl.loop(0, 4)
  def _(j):
    pltpu.async_copy(ones_v, deg_sh.at[dst_v.at[j]], dsem, add=True)

  @pl.loop(4, NCHUNK)
  def _(j):
    pltpu.async_copy(ones_v, deg_sh.at[dst_v.at[j]], dsem, add=True)
    pltpu.make_async_copy(ones_v, deg_sh.at[dst_v.at[j - 4]], dsem).wait()

  @pl.loop(NCHUNK - 4, NCHUNK)
  def _(j):
    pltpu.make_async_copy(ones_v, deg_sh.at[dst_v.at[j]], dsem).wait()

  plsc.subcore_barrier()

  @pl.when(s == 0)
  def _():
    pltpu.sync_copy(deg_sh, deg_out.at[c])


_deg_kernel = functools.partial(
    pl.kernel,
    out_type=jax.ShapeDtypeStruct((NC, N_ACC), jnp.float32),
    mesh=_MESH,
    scratch_types=[
        pltpu.VMEM((NCHUNK, ECH), jnp.int32),
        pltpu.VMEM((ECH,), jnp.float32),
        pltpu.VMEM_SHARED((N_ACC,), jnp.float32),
        pltpu.SemaphoreType.DMA,
    ],
)(_deg_body)


def _edge_body(g_hbm, src_hbm, dst_hbm, zrow_hbm, out_hbm,
               si, di, b0, b1, acc, is0, is1, is2, is3, g0, g1):
  c = lax.axis_index("c")
  s = lax.axis_index("s")
  w = s * NC + c
  base = w * EPW
  isems = (is0, is1, is2, is3)
  bufs = (b0, b1)
  gsems = (g0, g1)

  def fetch_idx(j, k):
    pltpu.async_copy(src_hbm.at[pl.ds(base + j * ECH, ECH)], si.at[k], isems[k])
    pltpu.async_copy(dst_hbm.at[pl.ds(base + j * ECH, ECH)], di.at[k], isems[k])

  def wait_idx(j, k):
    pltpu.make_async_copy(
        src_hbm.at[pl.ds(base + j * ECH, ECH)], si.at[k], isems[k]).wait()
    pltpu.make_async_copy(
        dst_hbm.at[pl.ds(base + j * ECH, ECH)], di.at[k], isems[k]).wait()

  def start_gather(k, p):
    pltpu.async_copy(g_hbm.at[si.at[k]], bufs[p], gsems[p])

  def wait_gather(k, p):
    pltpu.make_async_copy(g_hbm.at[si.at[k]], bufs[p], gsems[p]).wait()

  def scatter(k, p):
    pltpu.sync_copy(bufs[p], acc.at[di.at[k]], add=True)

  for k in range(4):
    fetch_idx(k, k)

  @pl.when(s < NS - 1)
  def _():
    pltpu.sync_copy(zrow_hbm.at[pl.ds(s * RPT, RPT)],
                    acc.at[pl.ds(s * RPT, RPT)])

  @pl.when(s == NS - 1)
  def _():
    pltpu.sync_copy(zrow_hbm.at[pl.ds((NS - 1) * RPT, RPT_LAST)],
                    acc.at[pl.ds((NS - 1) * RPT, RPT_LAST)])

  plsc.subcore_barrier()
  wait_idx(0, 0)
  start_gather(0, 0)
  wait_idx(1, 1)
  start_gather(1, 1)

  def step(c, k, p):
    wait_gather(k, p)
    scatter(k, p)
    k2 = (k + 2) % 4
    wait_idx(c + 2, k2)
    start_gather(k2, p)

    @pl.when(c + 4 < NCHUNK)
    def _():
      fetch_idx(c + 4, k)

  @pl.loop(0, NCHUNK - 3, step=4)
  def _(j):
    for k in range(4):
      step(j + k, k, k % 2)

  wait_gather(0, 0)
  scatter(0, 0)
  wait_idx(NCHUNK - 1, 2)
  start_gather(2, 0)
  wait_gather(1, 1)
  scatter(1, 1)
  wait_gather(2, 0)
  scatter(2, 0)

  plsc.subcore_barrier()

  @pl.when(s < NS - 1)
  def _():
    pltpu.sync_copy(acc.at[pl.ds(s * RPT, RPT)],
                    out_hbm.at[c, pl.ds(s * RPT, RPT)])

  @pl.when(s == NS - 1)
  def _():
    pltpu.sync_copy(acc.at[pl.ds((NS - 1) * RPT, RPT_LAST)],
                    out_hbm.at[c, pl.ds((NS - 1) * RPT, RPT_LAST)])


_edge_kernel = functools.partial(
    pl.kernel,
    out_type=jax.ShapeDtypeStruct((NC, N_NODES, CH), jnp.float32),
    mesh=_MESH,
    scratch_types=[
        pltpu.VMEM((4, ECH), jnp.int32),
        pltpu.VMEM((4, ECH), jnp.int32),
        pltpu.VMEM((ECH, CH), jnp.float32),
        pltpu.VMEM((ECH, CH), jnp.float32),
        pltpu.VMEM_SHARED((N_ACC, CH), jnp.float32),
        pltpu.SemaphoreType.DMA,
        pltpu.SemaphoreType.DMA,
        pltpu.SemaphoreType.DMA,
        pltpu.SemaphoreType.DMA,
        pltpu.SemaphoreType.DMA,
        pltpu.SemaphoreType.DMA,
    ],
)(_edge_body)


BM = 2000

def _dinv_block(deg_ref):
  dl = deg_ref[0]
  return lax.rsqrt(dl[0] + dl[1] + 1.0)


def _lin_body(deg_ref, x_ref, w_ref, g_ref):
  dinv = _dinv_block(deg_ref)
  h = jnp.dot(x_ref[...], w_ref[...], preferred_element_type=jnp.float32)
  g_ref[...] = h * dinv[:, None]


def _lin(deg2, x, W):
  return pl.pallas_call(
      _lin_body,
      grid=(N_NODES // BM,),
      in_specs=[
          pl.BlockSpec((1, NC, BM), lambda i: (i, 0, 0)),
          pl.BlockSpec((BM, CH), lambda i: (i, 0)),
          pl.BlockSpec((CH, CH), lambda i: (0, 0)),
      ],
      out_specs=pl.BlockSpec((BM, CH), lambda i: (i, 0)),
      out_shape=jax.ShapeDtypeStruct((N_NODES, CH), jnp.float32),
  )(deg2, x, W)


def _fin_body(deg_ref, p_ref, g_ref, b_ref, o_ref):
  dinv = _dinv_block(deg_ref)
  t = (p_ref[0] + p_ref[1] + g_ref[...]) * dinv[:, None] + b_ref[...]
  o_ref[...] = jnp.maximum(t, 0.0)


def _fin(deg2, P, g, b2):
  return pl.pallas_call(
      _fin_body,
      grid=(N_NODES // BM,),
      in_specs=[
          pl.BlockSpec((1, NC, BM), lambda i: (i, 0, 0)),
          pl.BlockSpec((NC, BM, CH), lambda i: (0, i, 0)),
          pl.BlockSpec((BM, CH), lambda i: (i, 0)),
          pl.BlockSpec((1, CH), lambda i: (0, 0)),
      ],
      out_specs=pl.BlockSpec((BM, CH), lambda i: (i, 0)),
      out_shape=jax.ShapeDtypeStruct((N_NODES, CH), jnp.float32),
  )(deg2, P, g, b2)


@jax.jit
def kernel(x, edge_index, W, b):
  npad = E_PAD - N_EDGES
  src = jnp.concatenate(
      [edge_index[0].astype(jnp.int32),
       jnp.arange(npad, dtype=jnp.int32) % N_NODES])
  dst = jnp.concatenate(
      [edge_index[1].astype(jnp.int32),
       N_NODES + (jnp.arange(npad, dtype=jnp.int32) % N_TRASH)])
  ones_c = jnp.ones((ECH,), jnp.float32)
  zeros_n = jnp.zeros((N_ACC,), jnp.float32)
  zrow = jnp.zeros((N_NODES, CH), jnp.float32)

  deg2 = _deg_kernel(dst.reshape(NW, NCHUNK, ECH), ones_c, zeros_n)[:, :N_NODES]
  deg2 = deg2.reshape(NC, N_NODES // BM, BM).transpose(1, 0, 2)
  g = _lin(deg2, x, W)
  P = _edge_kernel(g, src, dst, zrow)
  return _fin(deg2, P, g, b.reshape(1, CH))


# speedup vs baseline: 1.1240x; 1.0090x over previous
import functools

import jax
import jax.numpy as jnp
from jax import lax
from jax.experimental import pallas as pl
from jax.experimental.pallas import tpu as pltpu
from jax.experimental.pallas import tpu_sc as plsc

N_NODES = 10000
N_EDGES = 320000
CH = 128

NC = 2
NS = 16
NW = NC * NS
ECH = 128
NCHUNK = 79
EPW = NCHUNK * ECH
E_PAD = NW * EPW
N_TRASH = 64
N_ACC = 10240
RPT = 624
RPT_LAST = N_NODES - RPT * (NS - 1)

_MESH = plsc.VectorSubcoreMesh(
    core_axis_name="c", subcore_axis_name="s", num_cores=NC, num_subcores=NS)


def _deg_body(dst_hbm, ones_hbm, zeros_hbm, deg_out,
              dst_v, ones_v, deg_sh, dsem):
  c = lax.axis_index("c")
  s = lax.axis_index("s")
  w = s * NC + c

  @pl.when(s == 0)
  def _():
    pltpu.sync_copy(zeros_hbm, deg_sh)

  pltpu.sync_copy(dst_hbm.at[w], dst_v)
  pltpu.sync_copy(ones_hbm, ones_v)
  plsc.subcore_barrier()

  @pl.loop(0, 4)
  def _(j):
    pltpu.async_copy(ones_v, deg_sh.at[dst_v.at[j]], dsem, add=True)

  @pl.loop(4, NCHUNK)
  def _(j):
    pltpu.async_copy(ones_v, deg_sh.at[dst_v.at[j]], dsem, add=True)
    pltpu.make_async_copy(ones_v, deg_sh.at[dst_v.at[j - 4]], dsem).wait()

  @pl.loop(NCHUNK - 4, NCHUNK)
  def _(j):
    pltpu.make_async_copy(ones_v, deg_sh.at[dst_v.at[j]], dsem).wait()

  plsc.subcore_barrier()

  @pl.when(s == 0)
  def _():
    pltpu.sync_copy(deg_sh, deg_out.at[c])


_deg_kernel = functools.partial(
    pl.kernel,
    out_type=jax.ShapeDtypeStruct((NC, N_ACC), jnp.float32),
    mesh=_MESH,
    scratch_types=[
        pltpu.VMEM((NCHUNK, ECH), jnp.int32),
        pltpu.VMEM((ECH,), jnp.float32),
        pltpu.VMEM_SHARED((N_ACC,), jnp.float32),
        pltpu.SemaphoreType.DMA,
    ],
)(_deg_body)


def _edge_body(g_hbm, src_hbm, dst_hbm, zrow_hbm, out_hbm,
               si, di, b0, b1, acc, is0, is1, is2, is3, g0, g1):
  c = lax.axis_index("c")
  s = lax.axis_index("s")
  w = s * NC + c
  base = w * EPW
  isems = (is0, is1, is2, is3)
  bufs = (b0, b1)
  gsems = (g0, g1)

  def fetch_idx(j, k):
    pltpu.async_copy(src_hbm.at[pl.ds(base + j * ECH, ECH)], si.at[k], isems[k])
    pltpu.async_copy(dst_hbm.at[pl.ds(base + j * ECH, ECH)], di.at[k], isems[k])

  def wait_idx(j, k):
    pltpu.make_async_copy(
        src_hbm.at[pl.ds(base + j * ECH, ECH)], si.at[k], isems[k]).wait()
    pltpu.make_async_copy(
        dst_hbm.at[pl.ds(base + j * ECH, ECH)], di.at[k], isems[k]).wait()

  def start_gather(k, p):
    pltpu.async_copy(g_hbm.at[si.at[k]], bufs[p], gsems[p])

  def wait_gather(k, p):
    pltpu.make_async_copy(g_hbm.at[si.at[k]], bufs[p], gsems[p]).wait()

  def scatter(k, p):
    pltpu.sync_copy(bufs[p], acc.at[di.at[k]], add=True)

  for k in range(4):
    fetch_idx(k, k)

  @pl.when(s < NS - 1)
  def _():
    pltpu.sync_copy(zrow_hbm.at[pl.ds(s * RPT, RPT)],
                    acc.at[pl.ds(s * RPT, RPT)])

  @pl.when(s == NS - 1)
  def _():
    pltpu.sync_copy(zrow_hbm.at[pl.ds((NS - 1) * RPT, RPT_LAST)],
                    acc.at[pl.ds((NS - 1) * RPT, RPT_LAST)])

  plsc.subcore_barrier()
  wait_idx(0, 0)
  start_gather(0, 0)
  wait_idx(1, 1)
  start_gather(1, 1)

  def step(c, k, p):
    wait_gather(k, p)
    scatter(k, p)
    k2 = (k + 2) % 4
    wait_idx(c + 2, k2)
    start_gather(k2, p)

    @pl.when(c + 4 < NCHUNK)
    def _():
      fetch_idx(c + 4, k)

  @pl.loop(0, NCHUNK - 3, step=4)
  def _(j):
    for k in range(4):
      step(j + k, k, k % 2)

  wait_gather(0, 0)
  scatter(0, 0)
  wait_idx(NCHUNK - 1, 2)
  start_gather(2, 0)
  wait_gather(1, 1)
  scatter(1, 1)
  wait_gather(2, 0)
  scatter(2, 0)

  plsc.subcore_barrier()

  @pl.when(s < NS - 1)
  def _():
    pltpu.sync_copy(acc.at[pl.ds(s * RPT, RPT)],
                    out_hbm.at[c, pl.ds(s * RPT, RPT)])

  @pl.when(s == NS - 1)
  def _():
    pltpu.sync_copy(acc.at[pl.ds((NS - 1) * RPT, RPT_LAST)],
                    out_hbm.at[c, pl.ds((NS - 1) * RPT, RPT_LAST)])


_edge_kernel = functools.partial(
    pl.kernel,
    out_type=jax.ShapeDtypeStruct((NC, N_NODES, CH), jnp.float32),
    mesh=_MESH,
    scratch_types=[
        pltpu.VMEM((4, ECH), jnp.int32),
        pltpu.VMEM((4, ECH), jnp.int32),
        pltpu.VMEM((ECH, CH), jnp.float32),
        pltpu.VMEM((ECH, CH), jnp.float32),
        pltpu.VMEM_SHARED((N_ACC, CH), jnp.float32),
        pltpu.SemaphoreType.DMA,
        pltpu.SemaphoreType.DMA,
        pltpu.SemaphoreType.DMA,
        pltpu.SemaphoreType.DMA,
        pltpu.SemaphoreType.DMA,
        pltpu.SemaphoreType.DMA,
    ],
)(_edge_body)


BM = 2048

def _dinv_block(deg_ref):
  dl = deg_ref[...]
  return lax.rsqrt(dl[0] + dl[1] + 1.0)


def _lin_body(deg_ref, x_ref, w_ref, g_ref):
  dinv = _dinv_block(deg_ref)
  h = jnp.dot(x_ref[...], w_ref[...], preferred_element_type=jnp.float32)
  g_ref[...] = h * dinv[:, None]


def _lin(deg2, x, W):
  return pl.pallas_call(
      _lin_body,
      grid=(N_ACC // BM,),
      in_specs=[
          pl.BlockSpec((NC, BM), lambda i: (0, i)),
          pl.BlockSpec((BM, CH), lambda i: (i, 0)),
          pl.BlockSpec((CH, CH), lambda i: (0, 0)),
      ],
      out_specs=pl.BlockSpec((BM, CH), lambda i: (i, 0)),
      out_shape=jax.ShapeDtypeStruct((N_NODES, CH), jnp.float32),
  )(deg2, x, W)


def _fin_body(deg_ref, p_ref, g_ref, b_ref, o_ref):
  dinv = _dinv_block(deg_ref)
  t = (p_ref[0] + p_ref[1] + g_ref[...]) * dinv[:, None] + b_ref[...]
  o_ref[...] = jnp.maximum(t, 0.0)


def _fin(deg2, P, g, b2):
  return pl.pallas_call(
      _fin_body,
      grid=(N_ACC // BM,),
      in_specs=[
          pl.BlockSpec((NC, BM), lambda i: (0, i)),
          pl.BlockSpec((NC, BM, CH), lambda i: (0, i, 0)),
          pl.BlockSpec((BM, CH), lambda i: (i, 0)),
          pl.BlockSpec((1, CH), lambda i: (0, 0)),
      ],
      out_specs=pl.BlockSpec((BM, CH), lambda i: (i, 0)),
      out_shape=jax.ShapeDtypeStruct((N_NODES, CH), jnp.float32),
  )(deg2, P, g, b2)


@jax.jit
def kernel(x, edge_index, W, b):
  npad = E_PAD - N_EDGES
  src = jnp.concatenate(
      [edge_index[0].astype(jnp.int32),
       jnp.arange(npad, dtype=jnp.int32) % N_NODES])
  dst = jnp.concatenate(
      [edge_index[1].astype(jnp.int32),
       N_NODES + (jnp.arange(npad, dtype=jnp.int32) % N_TRASH)])
  ones_c = jnp.ones((ECH,), jnp.float32)
  zeros_n = jnp.zeros((N_ACC,), jnp.float32)
  zrow = jnp.zeros((N_NODES, CH), jnp.float32)

  deg2 = _deg_kernel(dst.reshape(NW, NCHUNK, ECH), ones_c, zeros_n)
  g = _lin(deg2, x, W)
  P = _edge_kernel(g, src, dst, zrow)
  return _fin(deg2, P, g, b.reshape(1, CH))


# speedup vs baseline: 1.1945x; 1.0627x over previous
import functools

import jax
import jax.numpy as jnp
from jax import lax
from jax.experimental import pallas as pl
from jax.experimental.pallas import tpu as pltpu
from jax.experimental.pallas import tpu_sc as plsc

N_NODES = 10000
N_EDGES = 320000
CH = 128

NC = 2
NS = 16
NW = NC * NS
ECH = 128
NCHUNK = 79
EPW = NCHUNK * ECH
E_PAD = NW * EPW
N_TRASH = 64
N_ACC = 10240
RPT = 624
RPT_LAST = N_NODES - RPT * (NS - 1)

_MESH = plsc.VectorSubcoreMesh(
    core_axis_name="c", subcore_axis_name="s", num_cores=NC, num_subcores=NS)


def _deg_body(edges_hbm, ones_hbm, zeros_hbm, deg_out,
              dst_v, ones_v, deg_sh, dsem, fsem):
  c = lax.axis_index("c")
  s = lax.axis_index("s")
  w = s * NC + c
  base = w * EPW

  def dfetch(j):
    pltpu.async_copy(
        edges_hbm.at[pl.ds(1, 1), pl.ds(base + j * ECH, ECH)],
        dst_v.at[pl.ds(j, 1)], fsem)

  def dfetch_wait(j):
    pltpu.make_async_copy(
        edges_hbm.at[pl.ds(1, 1), pl.ds(base + j * ECH, ECH)],
        dst_v.at[pl.ds(j, 1)], fsem).wait()

  def scat_start(j):
    pltpu.async_copy(ones_v, deg_sh.at[dst_v.at[j]], dsem, add=True)

  def scat_wait(j):
    pltpu.make_async_copy(ones_v, deg_sh.at[dst_v.at[j]], dsem).wait()

  @pl.when(s == 0)
  def _():
    pltpu.sync_copy(zeros_hbm, deg_sh)

  pltpu.sync_copy(ones_hbm, ones_v)

  @pl.loop(0, 8)
  def _(j):
    dfetch(j)

  plsc.subcore_barrier()

  @pl.loop(0, NCHUNK)
  def _(j):
    @pl.when(j + 8 < NCHUNK)
    def _():
      dfetch(j + 8)

    dfetch_wait(j)
    scat_start(j)

    @pl.when(j >= 4)
    def _():
      scat_wait(j - 4)

  @pl.loop(NCHUNK - 4, NCHUNK)
  def _(j):
    scat_wait(j)

  plsc.subcore_barrier()

  @pl.when(s == 0)
  def _():
    pltpu.sync_copy(deg_sh, deg_out.at[c])


_deg_kernel = functools.partial(
    pl.kernel,
    out_type=jax.ShapeDtypeStruct((NC, N_ACC), jnp.float32),
    mesh=_MESH,
    scratch_types=[
        pltpu.VMEM((NCHUNK, ECH), jnp.int32),
        pltpu.VMEM((ECH,), jnp.float32),
        pltpu.VMEM_SHARED((N_ACC,), jnp.float32),
        pltpu.SemaphoreType.DMA,
        pltpu.SemaphoreType.DMA,
    ],
)(_deg_body)


def _edge_body(g_hbm, edges_hbm, zrow_hbm, out_hbm,
               si, di, b0, b1, acc, is0, is1, is2, is3, g0, g1):
  c = lax.axis_index("c")
  s = lax.axis_index("s")
  w = s * NC + c
  base = w * EPW
  isems = (is0, is1, is2, is3)
  bufs = (b0, b1)
  gsems = (g0, g1)

  def fetch_idx(j, k):
    pltpu.async_copy(edges_hbm.at[pl.ds(0, 1), pl.ds(base + j * ECH, ECH)],
                     si.at[pl.ds(k, 1)], isems[k])
    pltpu.async_copy(edges_hbm.at[pl.ds(1, 1), pl.ds(base + j * ECH, ECH)],
                     di.at[pl.ds(k, 1)], isems[k])

  def wait_idx(j, k):
    pltpu.make_async_copy(
        edges_hbm.at[pl.ds(0, 1), pl.ds(base + j * ECH, ECH)],
        si.at[pl.ds(k, 1)], isems[k]).wait()
    pltpu.make_async_copy(
        edges_hbm.at[pl.ds(1, 1), pl.ds(base + j * ECH, ECH)],
        di.at[pl.ds(k, 1)], isems[k]).wait()

  def start_gather(k, p):
    pltpu.async_copy(g_hbm.at[si.at[k]], bufs[p], gsems[p])

  def wait_gather(k, p):
    pltpu.make_async_copy(g_hbm.at[si.at[k]], bufs[p], gsems[p]).wait()

  def scatter(k, p):
    pltpu.sync_copy(bufs[p], acc.at[di.at[k]], add=True)

  for k in range(4):
    fetch_idx(k, k)

  @pl.when(s < NS - 1)
  def _():
    pltpu.sync_copy(zrow_hbm.at[pl.ds(s * RPT, RPT)],
                    acc.at[pl.ds(s * RPT, RPT)])

  @pl.when(s == NS - 1)
  def _():
    pltpu.sync_copy(zrow_hbm.at[pl.ds((NS - 1) * RPT, RPT_LAST)],
                    acc.at[pl.ds((NS - 1) * RPT, RPT_LAST)])

  plsc.subcore_barrier()
  wait_idx(0, 0)
  start_gather(0, 0)
  wait_idx(1, 1)
  start_gather(1, 1)

  def step(c, k, p):
    wait_gather(k, p)
    scatter(k, p)
    k2 = (k + 2) % 4
    wait_idx(c + 2, k2)
    start_gather(k2, p)

    @pl.when(c + 4 < NCHUNK)
    def _():
      fetch_idx(c + 4, k)

  @pl.loop(0, NCHUNK - 3, step=4)
  def _(j):
    for k in range(4):
      step(j + k, k, k % 2)

  wait_gather(0, 0)
  scatter(0, 0)
  wait_idx(NCHUNK - 1, 2)
  start_gather(2, 0)
  wait_gather(1, 1)
  scatter(1, 1)
  wait_gather(2, 0)
  scatter(2, 0)

  plsc.subcore_barrier()

  @pl.when(s < NS - 1)
  def _():
    pltpu.sync_copy(acc.at[pl.ds(s * RPT, RPT)],
                    out_hbm.at[c, pl.ds(s * RPT, RPT)])

  @pl.when(s == NS - 1)
  def _():
    pltpu.sync_copy(acc.at[pl.ds((NS - 1) * RPT, RPT_LAST)],
                    out_hbm.at[c, pl.ds((NS - 1) * RPT, RPT_LAST)])


_edge_kernel = functools.partial(
    pl.kernel,
    out_type=jax.ShapeDtypeStruct((NC, N_NODES, CH), jnp.float32),
    mesh=_MESH,
    scratch_types=[
        pltpu.VMEM((4, ECH), jnp.int32),
        pltpu.VMEM((4, ECH), jnp.int32),
        pltpu.VMEM((ECH, CH), jnp.float32),
        pltpu.VMEM((ECH, CH), jnp.float32),
        pltpu.VMEM_SHARED((N_ACC, CH), jnp.float32),
        pltpu.SemaphoreType.DMA,
        pltpu.SemaphoreType.DMA,
        pltpu.SemaphoreType.DMA,
        pltpu.SemaphoreType.DMA,
        pltpu.SemaphoreType.DMA,
        pltpu.SemaphoreType.DMA,
    ],
)(_edge_body)


BM = 2048

def _dinv_block(deg_ref):
  dl = deg_ref[...]
  return lax.rsqrt(dl[0] + dl[1] + 1.0)


def _lin_body(deg_ref, x_ref, w_ref, g_ref):
  dinv = _dinv_block(deg_ref)
  h = jnp.dot(x_ref[...], w_ref[...], preferred_element_type=jnp.float32)
  g_ref[...] = h * dinv[:, None]


def _lin(deg2, x, W):
  return pl.pallas_call(
      _lin_body,
      grid=(N_ACC // BM,),
      in_specs=[
          pl.BlockSpec((NC, BM), lambda i: (0, i)),
          pl.BlockSpec((BM, CH), lambda i: (i, 0)),
          pl.BlockSpec((CH, CH), lambda i: (0, 0)),
      ],
      out_specs=pl.BlockSpec((BM, CH), lambda i: (i, 0)),
      out_shape=jax.ShapeDtypeStruct((N_NODES, CH), jnp.float32),
  )(deg2, x, W)


def _fin_body(deg_ref, p_ref, g_ref, b_ref, o_ref):
  dinv = _dinv_block(deg_ref)
  t = (p_ref[0] + p_ref[1] + g_ref[...]) * dinv[:, None] + b_ref[...]
  o_ref[...] = jnp.maximum(t, 0.0)


def _fin(deg2, P, g, b2):
  return pl.pallas_call(
      _fin_body,
      grid=(N_ACC // BM,),
      in_specs=[
          pl.BlockSpec((NC, BM), lambda i: (0, i)),
          pl.BlockSpec((NC, BM, CH), lambda i: (0, i, 0)),
          pl.BlockSpec((BM, CH), lambda i: (i, 0)),
          pl.BlockSpec((1, CH), lambda i: (0, 0)),
      ],
      out_specs=pl.BlockSpec((BM, CH), lambda i: (i, 0)),
      out_shape=jax.ShapeDtypeStruct((N_NODES, CH), jnp.float32),
  )(deg2, P, g, b2)


@jax.jit
def kernel(x, edge_index, W, b):
  npad = E_PAD - N_EDGES
  pads = jnp.stack([jnp.arange(npad, dtype=jnp.int32) % N_NODES,
                    N_NODES + (jnp.arange(npad, dtype=jnp.int32) % N_TRASH)])
  edges_p = jnp.concatenate([edge_index.astype(jnp.int32), pads], axis=1)
  ones_c = jnp.ones((ECH,), jnp.float32)
  zeros_n = jnp.zeros((N_ACC,), jnp.float32)
  zrow = jnp.zeros((N_NODES, CH), jnp.float32)

  deg2 = _deg_kernel(edges_p, ones_c, zeros_n)
  g = _lin(deg2, x, W)
  P = _edge_kernel(g, edges_p, zrow)
  return _fin(deg2, P, g, b.reshape(1, CH))
